# baseline (speedup 1.0000x reference)
import functools

import jax
import jax.numpy as jnp
from jax import lax
from jax.experimental import pallas as pl
from jax.experimental.pallas import tpu as pltpu
from jax.experimental.pallas import tpu_sc as plsc

N0, N1, N2 = 10000, 4000, 1024
E1, E2 = 160000, 32768
D = 128
NC, NS = 2, 16
NW = NC * NS
CH = 128

R1 = 4096
R2 = 1024
E1P = 163840
KA, KB = 40, 40


def _segsum_sc(table, src, dst, n_rows, n_tbl, k0, k1):
    rows_per_sub = n_rows // NS
    t0 = ((n_tbl // NS) // 8) * 8
    t_last = n_tbl - (NS - 1) * t0
    mesh = plsc.VectorSubcoreMesh(
        core_axis_name="c", subcore_axis_name="s",
        num_cores=NC, num_subcores=NS)

    @functools.partial(
        pl.kernel,
        out_type=jax.ShapeDtypeStruct((NC * n_rows, D), jnp.float32),
        mesh=mesh,
        scratch_types=[
            pltpu.VMEM((2, CH), jnp.int32),
            pltpu.VMEM((2, CH), jnp.int32),
            pltpu.VMEM((2, CH, D), jnp.float32),
            pltpu.VMEM_SHARED((n_tbl, D), jnp.float32),
            pltpu.VMEM_SHARED((n_rows, D), jnp.float32),
            pltpu.SemaphoreType.DMA,
            pltpu.SemaphoreType.DMA,
            pltpu.SemaphoreType.DMA,
            pltpu.SemaphoreType.DMA,
        ],
    )
    def k(table_hbm, src_hbm, dst_hbm, zero_hbm, out_hbm,
          sidx, didx, rows, tbl, acc, si0, si1, sg0, sg1):
        si = (si0, si1)
        sg = (sg0, sg1)
        cid = lax.axis_index("c")
        sid = lax.axis_index("s")
        rbase = sid * rows_per_sub

        @pl.when(sid < NS - 1)
        def _():
            pltpu.sync_copy(table_hbm.at[pl.ds(sid * t0, t0)],
                            tbl.at[pl.ds(sid * t0, t0)])

        @pl.when(sid == NS - 1)
        def _():
            pltpu.sync_copy(table_hbm.at[pl.ds((NS - 1) * t0, t_last)],
                            tbl.at[pl.ds((NS - 1) * t0, t_last)])

        pltpu.sync_copy(zero_hbm.at[pl.ds(rbase, rows_per_sub)],
                        acc.at[pl.ds(rbase, rows_per_sub)])
        plsc.subcore_barrier()

        def wait_gather(p):
            pltpu.make_async_copy(tbl.at[sidx.at[p]], rows.at[p],
                                  sg[p]).wait()

        def stage(cbase, j, p):
            off = (cbase + j) * CH
            d1 = pltpu.async_copy(src_hbm.at[pl.ds(off, CH)], sidx.at[p],
                                  si[p])
            d2 = pltpu.async_copy(dst_hbm.at[pl.ds(off, CH)], didx.at[p],
                                  si[p])
            d1.wait()
            d2.wait()
            pltpu.async_copy(tbl.at[sidx.at[p]], rows.at[p], sg[p])

        def drain(p):
            wait_gather(p)
            pltpu.sync_copy(rows.at[p], acc.at[didx.at[p]], add=True)

        def run_chunks(cbase, n):
            n2 = n // 2
            stage(cbase, 0, 0)
            stage(cbase, 1, 1)

            def body(i, carry):
                drain(0)

                @pl.when(i < n2 - 1)
                def _():
                    stage(cbase, 2 * i + 2, 0)

                drain(1)

                @pl.when(i < n2 - 1)
                def _():
                    stage(cbase, 2 * i + 3, 1)

                return carry

            lax.fori_loop(0, n2, body, 0)

        if k0 == k1:
            run_chunks((cid * NS + sid) * k0, k0)
        else:
            @pl.when(cid == 0)
            def _():
                run_chunks(sid * k0, k0)

            @pl.when(cid == 1)
            def _():
                run_chunks(NS * k0 + sid * k1, k1)

        plsc.subcore_barrier()
        pltpu.sync_copy(acc.at[pl.ds(rbase, rows_per_sub)],
                        out_hbm.at[pl.ds(cid * n_rows + rbase, rows_per_sub)])

    zeros = jnp.zeros((n_rows, D), jnp.float32)
    return k(table, src, dst, zeros)


def _deg_sc(dst1p, dst2):
    rps1 = R1 // NS
    rps2 = R2 // NS
    kk1 = E1P // CH // NW
    kk2 = E2 // CH // NW
    mesh = plsc.VectorSubcoreMesh(
        core_axis_name="c", subcore_axis_name="s",
        num_cores=NC, num_subcores=NS)

    @functools.partial(
        pl.kernel,
        out_type=[jax.ShapeDtypeStruct((NC * R1, D), jnp.float32),
                  jax.ShapeDtypeStruct((NC * R2, D), jnp.float32)],
        mesh=mesh,
        scratch_types=[
            pltpu.VMEM((4, CH), jnp.int32),
            pltpu.VMEM((CH, D), jnp.float32),
            pltpu.VMEM_SHARED((R1, D), jnp.float32),
            pltpu.VMEM_SHARED((R2, D), jnp.float32),
            pltpu.SemaphoreType.DMA,
            pltpu.SemaphoreType.DMA,
            pltpu.SemaphoreType.DMA,
            pltpu.SemaphoreType.DMA,
            pltpu.SemaphoreType.DMA,
            pltpu.SemaphoreType.DMA,
            pltpu.SemaphoreType.DMA,
            pltpu.SemaphoreType.DMA,
        ],
    )
    def k(dst1_hbm, dst2_hbm, zero_hbm, ones_hbm, deg1_hbm, deg2_hbm,
          didx, onesbuf, dacc1, dacc2, si0, si1, si2, si3,
          ss0, ss1, ss2, ss3):
        si = (si0, si1, si2, si3)
        ss = (ss0, ss1, ss2, ss3)
        cid = lax.axis_index("c")
        sid = lax.axis_index("s")
        wid = cid * NS + sid
        pltpu.sync_copy(zero_hbm.at[pl.ds(sid * rps1, rps1)],
                        dacc1.at[pl.ds(sid * rps1, rps1)])
        pltpu.sync_copy(zero_hbm.at[pl.ds(sid * rps2, rps2)],
                        dacc2.at[pl.ds(sid * rps2, rps2)])
        pltpu.sync_copy(ones_hbm, onesbuf)
        plsc.subcore_barrier()

        def run(dst_hbm, dacc, cbase, n):
            def issue(j, p):
                off = (cbase + j) * CH
                pltpu.async_copy(dst_hbm.at[pl.ds(off, CH)], didx.at[p],
                                 si[p])

            def wait_idx(p):
                pltpu.make_async_copy(dst_hbm.at[pl.ds(0, CH)],
                                      didx.at[p], si[p]).wait()

            def wait_scatter(p):
                pltpu.make_async_copy(onesbuf, dacc.at[didx.at[p]],
                                      ss[p]).wait()

            n4 = n // 4
            for p in range(4):
                issue(p, p)

            def body(i, carry):
                for p in range(4):
                    wait_idx(p)
                    pltpu.async_copy(onesbuf, dacc.at[didx.at[p]], ss[p],
                                     add=True)

                @pl.when(i < n4 - 1)
                def _():
                    for p in range(4):
                        wait_scatter(p)
                        issue(4 * i + 4 + p, p)

                return carry

            lax.fori_loop(0, n4, body, 0)
            for p in range(4):
                wait_scatter(p)

        run(dst1_hbm, dacc1, wid * kk1, kk1)
        run(dst2_hbm, dacc2, wid * kk2, kk2)
        plsc.subcore_barrier()
        pltpu.sync_copy(dacc1.at[pl.ds(sid * rps1, rps1)],
                        deg1_hbm.at[pl.ds(cid * R1 + sid * rps1, rps1)])
        pltpu.sync_copy(dacc2.at[pl.ds(sid * rps2, rps2)],
                        deg2_hbm.at[pl.ds(cid * R2 + sid * rps2, rps2)])

    zeros = jnp.zeros((R1, D), jnp.float32)
    ones = jnp.ones((CH, D), jnp.float32)
    return k(dst1p, dst2, zeros, ones)


def _deg_col(degp, n_rows, n):
    d = degp[:, 0:1]
    return (d[:n_rows] + d[n_rows:])[:n]


def _pad_rows(v, rows):
    return jnp.concatenate(
        [v, jnp.zeros((rows - v.shape[0], v.shape[1]), v.dtype)], axis=0)


def _tc_prep_body(x_ref, xb_ref, xtb_ref):
    xs = x_ref[:]
    xt = xs[:N1]
    m_all = jnp.mean(xs, axis=0, keepdims=True)
    m_tgt = jnp.mean(xt, axis=0, keepdims=True)
    xb_ref[:] = _pad_rows(jnp.sign(xt - m_all), R1)
    xtb_ref[:] = jnp.sign(xt - m_tgt)


def _tc_l1a_body(segp_ref, degp_ref, xtb_ref, w_ref, b_ref, h_ref):
    sp = segp_ref[:]
    seg = (sp[:R1] + sp[R1:])[:N1]
    deg = _deg_col(degp_ref[:], R1, N1)
    agg = (seg + xtb_ref[:]) / (deg + 1.0)
    h = jnp.dot(agg, w_ref[:], preferred_element_type=jnp.float32) + b_ref[:]
    h_ref[:] = _pad_rows(h, R1)


def _tc_l1b_body(seg2p_ref, degp_ref, h_ref, w_ref, b_ref,
                 xb2_ref, xtb2_ref):
    s2 = (seg2p_ref[:][:R1] + seg2p_ref[:][R1:])[:N1]
    deg = _deg_col(degp_ref[:], R1, N1)
    h = h_ref[:N1]
    agg2 = (s2 + h) / (deg + 1.0)
    h2 = jnp.maximum(
        jnp.dot(agg2, w_ref[:], preferred_element_type=jnp.float32)
        + b_ref[:], 0.0)
    ht = h2[:N2]
    m_all = jnp.mean(h2, axis=0, keepdims=True)
    m_tgt = jnp.mean(ht, axis=0, keepdims=True)
    xb2_ref[:] = jnp.sign(ht - m_all)
    xtb2_ref[:] = jnp.sign(ht - m_tgt)


def _tc_l2a_body(seg3p_ref, degp_ref, xtb2_ref, w_ref, b_ref, h3_ref):
    sp = seg3p_ref[:]
    seg = sp[:R2] + sp[R2:]
    deg = _deg_col(degp_ref[:], R2, N2)
    agg = (seg + xtb2_ref[:]) / (deg + 1.0)
    h3_ref[:] = jnp.dot(agg, w_ref[:],
                        preferred_element_type=jnp.float32) + b_ref[:]


def _tc_l2b_body(seg4p_ref, degp_ref, h3_ref, w_ref, b_ref, out_ref):
    s4 = seg4p_ref[:][:R2] + seg4p_ref[:][R2:]
    deg = _deg_col(degp_ref[:], R2, N2)
    h3 = h3_ref[:]
    agg2 = (s4 + h3) / (deg + 1.0)
    o = jnp.dot(agg2, w_ref[:], preferred_element_type=jnp.float32) + b_ref[:]
    m = jnp.max(o, axis=-1, keepdims=True)
    ls = o - m
    out_ref[:] = ls - jnp.log(jnp.sum(jnp.exp(ls), axis=-1, keepdims=True))


def kernel(x, edge_index1, edge_index2, W1a, b1a, W1b, b1b, W2a, b2a, W2b,
           b2b):
    src1, dst1 = edge_index1[0], edge_index1[1]
    src2, dst2 = edge_index2[0], edge_index2[1]
    npad = E1P - E1
    src1p = jnp.concatenate([src1, jnp.zeros((npad,), jnp.int32)])
    dst1p = jnp.concatenate([dst1, jnp.full((npad,), N1, jnp.int32)])

    k2 = E2 // CH // NW

    deg1p, deg2p = _deg_sc(dst1p, dst2)

    xb, xtb = pl.pallas_call(
        _tc_prep_body,
        out_shape=[jax.ShapeDtypeStruct((R1, D), jnp.float32),
                   jax.ShapeDtypeStruct((N1, D), jnp.float32)],
    )(x)

    seg1p = _segsum_sc(xb, src1p, dst1p, R1, N1, KA, KB)

    h = pl.pallas_call(
        _tc_l1a_body,
        out_shape=jax.ShapeDtypeStruct((R1, D), jnp.float32),
    )(seg1p, deg1p, xtb, W1a, b1a)

    seg2p = _segsum_sc(h, src1p, dst1p, R1, N1, KA, KB)

    xb2, xtb2 = pl.pallas_call(
        _tc_l1b_body,
        out_shape=[jax.ShapeDtypeStruct((R2, D), jnp.float32),
                   jax.ShapeDtypeStruct((N2, D), jnp.float32)],
    )(seg2p, deg1p, h, W1b, b1b)

    seg3p = _segsum_sc(xb2, src2, dst2, R2, N2, k2, k2)

    h3 = pl.pallas_call(
        _tc_l2a_body,
        out_shape=jax.ShapeDtypeStruct((R2, D), jnp.float32),
    )(seg3p, deg2p, xtb2, W2a, b2a)

    seg4p = _segsum_sc(h3, src2, dst2, R2, N2, k2, k2)

    out = pl.pallas_call(
        _tc_l2b_body,
        out_shape=jax.ShapeDtypeStruct((N2, D), jnp.float32),
    )(seg4p, deg2p, h3, W2b, b2b)
    return out


# speedup vs baseline: 1.0109x; 1.0109x over previous
import functools

import jax
import jax.numpy as jnp
from jax import lax
from jax.experimental import pallas as pl
from jax.experimental.pallas import tpu as pltpu
from jax.experimental.pallas import tpu_sc as plsc

N0, N1, N2 = 10000, 4000, 1024
E1, E2 = 160000, 32768
D = 128
NC, NS = 2, 16
NW = NC * NS
CH = 128

R1 = 4096
R2 = 1024
E1P = 163840
KA, KB = 40, 40


def _segsum_sc(table, src, dst, n_rows, n_tbl, k0, k1):
    rows_per_sub = n_rows // NS
    t0 = ((n_tbl // NS) // 8) * 8
    t_last = n_tbl - (NS - 1) * t0
    mesh = plsc.VectorSubcoreMesh(
        core_axis_name="c", subcore_axis_name="s",
        num_cores=NC, num_subcores=NS)

    @functools.partial(
        pl.kernel,
        out_type=jax.ShapeDtypeStruct((NC * n_rows, D), jnp.float32),
        mesh=mesh,
        scratch_types=[
            pltpu.VMEM((2, CH), jnp.int32),
            pltpu.VMEM((2, CH), jnp.int32),
            pltpu.VMEM((2, CH, D), jnp.float32),
            pltpu.VMEM_SHARED((n_tbl, D), jnp.float32),
            pltpu.VMEM_SHARED((n_rows, D), jnp.float32),
            pltpu.SemaphoreType.DMA,
            pltpu.SemaphoreType.DMA,
            pltpu.SemaphoreType.DMA,
            pltpu.SemaphoreType.DMA,
        ],
    )
    def k(table_hbm, src_hbm, dst_hbm, zero_hbm, out_hbm,
          sidx, didx, rows, tbl, acc, si0, si1, sg0, sg1):
        si = (si0, si1)
        sg = (sg0, sg1)
        cid = lax.axis_index("c")
        sid = lax.axis_index("s")
        rbase = sid * rows_per_sub

        @pl.when(sid < NS - 1)
        def _():
            pltpu.sync_copy(table_hbm.at[pl.ds(sid * t0, t0)],
                            tbl.at[pl.ds(sid * t0, t0)])

        @pl.when(sid == NS - 1)
        def _():
            pltpu.sync_copy(table_hbm.at[pl.ds((NS - 1) * t0, t_last)],
                            tbl.at[pl.ds((NS - 1) * t0, t_last)])

        pltpu.sync_copy(zero_hbm.at[pl.ds(rbase, rows_per_sub)],
                        acc.at[pl.ds(rbase, rows_per_sub)])
        plsc.subcore_barrier()

        def wait_gather(p):
            pltpu.make_async_copy(tbl.at[sidx.at[p]], rows.at[p],
                                  sg[p]).wait()

        def stage(cbase, j, p):
            off = (cbase + j) * CH
            d1 = pltpu.async_copy(src_hbm.at[pl.ds(off, CH)], sidx.at[p],
                                  si[p])
            d2 = pltpu.async_copy(dst_hbm.at[pl.ds(off, CH)], didx.at[p],
                                  si[p])
            d1.wait()
            d2.wait()
            pltpu.async_copy(tbl.at[sidx.at[p]], rows.at[p], sg[p])

        def drain(p):
            wait_gather(p)
            pltpu.sync_copy(rows.at[p], acc.at[didx.at[p]], add=True)

        def run_chunks(cbase, n):
            n2 = n // 2
            stage(cbase, 0, 0)
            stage(cbase, 1, 1)

            def body(i, carry):
                drain(0)

                @pl.when(i < n2 - 1)
                def _():
                    stage(cbase, 2 * i + 2, 0)

                drain(1)

                @pl.when(i < n2 - 1)
                def _():
                    stage(cbase, 2 * i + 3, 1)

                return carry

            lax.fori_loop(0, n2, body, 0)

        if k0 == k1:
            run_chunks((cid * NS + sid) * k0, k0)
        else:
            @pl.when(cid == 0)
            def _():
                run_chunks(sid * k0, k0)

            @pl.when(cid == 1)
            def _():
                run_chunks(NS * k0 + sid * k1, k1)

        plsc.subcore_barrier()
        pltpu.sync_copy(acc.at[pl.ds(rbase, rows_per_sub)],
                        out_hbm.at[pl.ds(cid * n_rows + rbase, rows_per_sub)])

    zeros = jnp.zeros((n_rows, D), jnp.float32)
    return k(table, src, dst, zeros)


def _deg_sc(dst1p, dst2):
    rps1 = R1 // NS
    rps2 = R2 // NS
    kk1 = E1P // CH // NW
    kk2 = E2 // CH // NW
    mesh = plsc.VectorSubcoreMesh(
        core_axis_name="c", subcore_axis_name="s",
        num_cores=NC, num_subcores=NS)

    @functools.partial(
        pl.kernel,
        out_type=[jax.ShapeDtypeStruct((NC * R1, D), jnp.float32),
                  jax.ShapeDtypeStruct((NC * R2, D), jnp.float32)],
        mesh=mesh,
        scratch_types=[
            pltpu.VMEM((4, CH), jnp.int32),
            pltpu.VMEM((CH, D), jnp.float32),
            pltpu.VMEM_SHARED((R1, D), jnp.float32),
            pltpu.VMEM_SHARED((R2, D), jnp.float32),
            pltpu.SemaphoreType.DMA,
            pltpu.SemaphoreType.DMA,
            pltpu.SemaphoreType.DMA,
            pltpu.SemaphoreType.DMA,
            pltpu.SemaphoreType.DMA,
            pltpu.SemaphoreType.DMA,
            pltpu.SemaphoreType.DMA,
            pltpu.SemaphoreType.DMA,
        ],
    )
    def k(dst1_hbm, dst2_hbm, zero_hbm, ones_hbm, deg1_hbm, deg2_hbm,
          didx, onesbuf, dacc1, dacc2, si0, si1, si2, si3,
          ss0, ss1, ss2, ss3):
        si = (si0, si1, si2, si3)
        ss = (ss0, ss1, ss2, ss3)
        cid = lax.axis_index("c")
        sid = lax.axis_index("s")
        wid = cid * NS + sid
        pltpu.sync_copy(zero_hbm.at[pl.ds(sid * rps1, rps1)],
                        dacc1.at[pl.ds(sid * rps1, rps1)])
        pltpu.sync_copy(zero_hbm.at[pl.ds(sid * rps2, rps2)],
                        dacc2.at[pl.ds(sid * rps2, rps2)])
        pltpu.sync_copy(ones_hbm, onesbuf)
        plsc.subcore_barrier()

        def run(dst_hbm, dacc, cbase, n):
            def issue(j, p):
                off = (cbase + j) * CH
                pltpu.async_copy(dst_hbm.at[pl.ds(off, CH)], didx.at[p],
                                 si[p])

            def wait_idx(p):
                pltpu.make_async_copy(dst_hbm.at[pl.ds(0, CH)],
                                      didx.at[p], si[p]).wait()

            def wait_scatter(p):
                pltpu.make_async_copy(onesbuf, dacc.at[didx.at[p]],
                                      ss[p]).wait()

            n4 = n // 4
            for p in range(4):
                issue(p, p)

            def body(i, carry):
                for p in range(4):
                    wait_idx(p)
                    pltpu.async_copy(onesbuf, dacc.at[didx.at[p]], ss[p],
                                     add=True)

                @pl.when(i < n4 - 1)
                def _():
                    for p in range(4):
                        wait_scatter(p)
                        issue(4 * i + 4 + p, p)

                return carry

            lax.fori_loop(0, n4, body, 0)
            for p in range(4):
                wait_scatter(p)

        run(dst1_hbm, dacc1, wid * kk1, kk1)
        run(dst2_hbm, dacc2, wid * kk2, kk2)
        plsc.subcore_barrier()
        pltpu.sync_copy(dacc1.at[pl.ds(sid * rps1, rps1)],
                        deg1_hbm.at[pl.ds(cid * R1 + sid * rps1, rps1)])
        pltpu.sync_copy(dacc2.at[pl.ds(sid * rps2, rps2)],
                        deg2_hbm.at[pl.ds(cid * R2 + sid * rps2, rps2)])

    zeros = jnp.zeros((R1, D), jnp.float32)
    ones = jnp.ones((CH, D), jnp.float32)
    return k(dst1p, dst2, zeros, ones)


def _deg_col(degp, n_rows, n):
    d = degp[:, 0:1]
    return (d[:n_rows] + d[n_rows:])[:n]


def _pad_rows(v, rows):
    return jnp.concatenate(
        [v, jnp.zeros((rows - v.shape[0], v.shape[1]), v.dtype)], axis=0)


def _tc_prep_body(x_ref, xb_ref, xtb_ref):
    xs = x_ref[:]
    xt = xs[:N1]
    m_all = jnp.mean(xs, axis=0, keepdims=True)
    m_tgt = jnp.mean(xt, axis=0, keepdims=True)
    xb_ref[:] = _pad_rows(jnp.sign(xt - m_all), R1)
    xtb_ref[:] = jnp.sign(xt - m_tgt)


def _tc_l1a_body(segp_ref, degp_ref, xtb_ref, agg_ref):
    sp = segp_ref[:]
    seg = (sp[:R1] + sp[R1:])[:N1]
    deg = _deg_col(degp_ref[:], R1, N1)
    agg = (seg + xtb_ref[:]) / (deg + 1.0)
    agg_ref[:] = _pad_rows(agg, R1)


def _tc_l1b_body(seg2p_ref, degp_ref, agg_ref, wa_ref, ba_ref, wb_ref,
                 bb_ref, xb2_ref, xtb2_ref):
    s2 = (seg2p_ref[:][:R1] + seg2p_ref[:][R1:])[:N1]
    deg = _deg_col(degp_ref[:], R1, N1)
    agg = agg_ref[:N1]
    agg2 = jnp.dot((s2 + agg) / (deg + 1.0), wa_ref[:],
                   preferred_element_type=jnp.float32) + ba_ref[:]
    h2 = jnp.maximum(
        jnp.dot(agg2, wb_ref[:], preferred_element_type=jnp.float32)
        + bb_ref[:], 0.0)
    ht = h2[:N2]
    m_all = jnp.mean(h2, axis=0, keepdims=True)
    m_tgt = jnp.mean(ht, axis=0, keepdims=True)
    xb2_ref[:] = jnp.sign(ht - m_all)
    xtb2_ref[:] = jnp.sign(ht - m_tgt)


def _tc_l2a_body(seg3p_ref, degp_ref, xtb2_ref, agg3_ref):
    sp = seg3p_ref[:]
    seg = sp[:R2] + sp[R2:]
    deg = _deg_col(degp_ref[:], R2, N2)
    agg3_ref[:] = (seg + xtb2_ref[:]) / (deg + 1.0)


def _tc_l2b_body(seg4p_ref, degp_ref, agg3_ref, wa_ref, ba_ref, wb_ref,
                 bb_ref, out_ref):
    s4 = seg4p_ref[:][:R2] + seg4p_ref[:][R2:]
    deg = _deg_col(degp_ref[:], R2, N2)
    agg3 = agg3_ref[:]
    agg2 = jnp.dot((s4 + agg3) / (deg + 1.0), wa_ref[:],
                   preferred_element_type=jnp.float32) + ba_ref[:]
    o = jnp.dot(agg2, wb_ref[:],
                preferred_element_type=jnp.float32) + bb_ref[:]
    m = jnp.max(o, axis=-1, keepdims=True)
    ls = o - m
    out_ref[:] = ls - jnp.log(jnp.sum(jnp.exp(ls), axis=-1, keepdims=True))


def kernel(x, edge_index1, edge_index2, W1a, b1a, W1b, b1b, W2a, b2a, W2b,
           b2b):
    src1, dst1 = edge_index1[0], edge_index1[1]
    src2, dst2 = edge_index2[0], edge_index2[1]
    npad = E1P - E1
    src1p = jnp.concatenate([src1, jnp.zeros((npad,), jnp.int32)])
    dst1p = jnp.concatenate([dst1, jnp.full((npad,), N1, jnp.int32)])

    k2 = E2 // CH // NW

    deg1p, deg2p = _deg_sc(dst1p, dst2)

    xb, xtb = pl.pallas_call(
        _tc_prep_body,
        out_shape=[jax.ShapeDtypeStruct((R1, D), jnp.float32),
                   jax.ShapeDtypeStruct((N1, D), jnp.float32)],
    )(x)

    seg1p = _segsum_sc(xb, src1p, dst1p, R1, N1, KA, KB)

    agg = pl.pallas_call(
        _tc_l1a_body,
        out_shape=jax.ShapeDtypeStruct((R1, D), jnp.float32),
    )(seg1p, deg1p, xtb)

    seg2p = _segsum_sc(agg, src1p, dst1p, R1, N1, KA, KB)

    xb2, xtb2 = pl.pallas_call(
        _tc_l1b_body,
        out_shape=[jax.ShapeDtypeStruct((R2, D), jnp.float32),
                   jax.ShapeDtypeStruct((N2, D), jnp.float32)],
    )(seg2p, deg1p, agg, W1a, b1a, W1b, b1b)

    seg3p = _segsum_sc(xb2, src2, dst2, R2, N2, k2, k2)

    agg3 = pl.pallas_call(
        _tc_l2a_body,
        out_shape=jax.ShapeDtypeStruct((R2, D), jnp.float32),
    )(seg3p, deg2p, xtb2)

    seg4p = _segsum_sc(agg3, src2, dst2, R2, N2, k2, k2)

    out = pl.pallas_call(
        _tc_l2b_body,
        out_shape=jax.ShapeDtypeStruct((N2, D), jnp.float32),
    )(seg4p, deg2p, agg3, W2a, b2a, W2b, b2b)
    return out
